# baseline (speedup 1.0000x reference)
import jax
import jax.numpy as jnp
from jax import lax
from jax.experimental import pallas as pl
from jax.experimental.pallas import tpu as pltpu
from jax.experimental.pallas import tpu_sc as plsc

D = 64
T = 200
B = 4096
V = 1000000
NC, NS = 2, 16
NW = NC * NS
LANES = 16

ROWS = B * T
ROWS_PER_W = ROWS // NW
CHUNK = 128
NCHUNK = ROWS_PER_W // CHUNK
NSLOT = 4
POS_ROWS = T + CHUNK
TBLK = 8192


def _relayout_body(src_ref, dst_ref):
    eye = (lax.broadcasted_iota(jnp.int32, (D, D), 0)
           == lax.broadcasted_iota(jnp.int32, (D, D), 1)).astype(jnp.float32)
    rows = lax.dot_general(
        src_ref[...], eye,
        dimension_numbers=(((0,), (0,)), ((), ())),
        preferred_element_type=jnp.float32,
    )
    dst_ref[...] = jnp.concatenate(
        [rows[0:TBLK // 2], rows[TBLK // 2:TBLK]], axis=1
    )


def _make_emb_body(nchunk, t_h, rows_per_w):
    def _emb_body(xw_hbm, tok_hbm, pos_hbm, out_hbm, idx_v, stage_v, pos_v,
                  gsem, osem):
        sid = lax.axis_index("s")
        wid = sid * NC + lax.axis_index("c")
        pltpu.sync_copy(xw_hbm.at[pl.ds(wid * nchunk, nchunk)], idx_v)

        @pl.when(sid == 0)
        def _():
            pltpu.sync_copy(pos_hbm, pos_v)

        plsc.subcore_barrier()
        row0 = wid * rows_per_w

        def prefill(c):
            phase = lax.rem(c * CHUNK, t_h)
            pltpu.sync_copy(pos_v.at[pl.ds(phase, CHUNK)],
                            stage_v.at[lax.rem(c, NSLOT)])

        def fire_gadd(c):
            slot = lax.rem(c, NSLOT)
            pltpu.async_copy(tok_hbm.at[idx_v.at[c]], stage_v.at[slot],
                             gsem.at[slot], add=True)

        def write_desc(c):
            slot = lax.rem(c, NSLOT)
            return pltpu.make_async_copy(
                stage_v.at[slot],
                out_hbm.at[pl.ds(row0 + c * CHUNK, CHUNK)],
                osem.at[slot],
            )

        for c in range(3):
            prefill(c)
        for c in range(2):
            fire_gadd(c)

        def chunk_body(c, carry):
            slot = lax.rem(c, NSLOT)
            pltpu.make_async_copy(tok_hbm.at[idx_v.at[c]], stage_v.at[slot],
                                  gsem.at[slot]).wait()
            write_desc(c).start()

            @pl.when(c >= 1)
            def _():
                write_desc(c - 1).wait()

            @pl.when(c + 3 < nchunk)
            def _():
                prefill(c + 3)

            @pl.when(c + 2 < nchunk)
            def _():
                fire_gadd(c + 2)

            return carry

        lax.fori_loop(0, nchunk, chunk_body, 0, unroll=False)
        write_desc(nchunk - 1).wait()

    return _emb_body


@jax.jit
def kernel(x, token_table, pos_table):
    nblk = pl.cdiv(V, TBLK)
    h = TBLK // 2
    xi = x.astype(jnp.int32)
    k = xi & (TBLK - 1)
    pi = (xi & ~(TBLK - 1)) + 2 * (k & (h - 1)) + (k >= h)
    tok2 = pl.pallas_call(
        _relayout_body,
        grid=(nblk,),
        in_specs=[pl.BlockSpec((D, TBLK), lambda j: (0, j))],
        out_specs=pl.BlockSpec((TBLK // 2, 128), lambda j: (j, 0)),
        out_shape=jax.ShapeDtypeStruct((nblk * (TBLK // 2), 128),
                                       jnp.float32),
    )(token_table.T)
    tok3 = tok2.reshape(nblk * TBLK, D)
    t_h = T // 2
    rows_h = B * t_h
    nchunk_h = rows_h // NW // CHUNK
    pos_rows_h = t_h + CHUNK
    mesh = plsc.VectorSubcoreMesh(core_axis_name="c", subcore_axis_name="s")
    halves = []
    for p in range(2):
        xw = pi[:, p * t_h:(p + 1) * t_h].reshape(rows_h // CHUNK, CHUNK)
        rr = jnp.arange(pos_rows_h) % t_h
        pos2 = pos_table[p * t_h + rr]
        run = pl.kernel(
            _make_emb_body(nchunk_h, t_h, rows_h // NW),
            mesh=mesh,
            compiler_params=pltpu.CompilerParams(use_tc_tiling_on_sc=False),
            out_type=jax.ShapeDtypeStruct((rows_h, D), jnp.float32),
            scratch_types=[
                pltpu.VMEM((nchunk_h, CHUNK), jnp.int32),
                pltpu.VMEM((NSLOT, CHUNK, D), jnp.float32),
                pltpu.VMEM_SHARED((pos_rows_h, D), jnp.float32),
                pltpu.SemaphoreType.DMA((NSLOT,)),
                pltpu.SemaphoreType.DMA((NSLOT,)),
            ],
        )
        halves.append(run(xw, tok3, pos2).reshape(B, t_h, D))
    return jnp.concatenate(halves, axis=1)


# speedup vs baseline: 2.0251x; 2.0251x over previous
import jax
import jax.numpy as jnp
from jax import lax
from jax.experimental import pallas as pl
from jax.experimental.pallas import tpu as pltpu
from jax.experimental.pallas import tpu_sc as plsc

D = 64
T = 200
B = 4096
V = 1000000
NC, NS = 2, 16
NW = NC * NS
LANES = 16

ROWS = B * T
ROWS_PER_W = ROWS // NW
CHUNK = 128
NCHUNK = ROWS_PER_W // CHUNK
NSLOT = 6
POS_ROWS = T + CHUNK
TBLK = 16384


def _relayout_body(src_ref, dst_ref):
    eye = (lax.broadcasted_iota(jnp.int32, (D, D), 0)
           == lax.broadcasted_iota(jnp.int32, (D, D), 1)).astype(jnp.float32)
    rows = lax.dot_general(
        src_ref[...], eye,
        dimension_numbers=(((0,), (0,)), ((), ())),
        preferred_element_type=jnp.float32,
    )
    dst_ref[...] = jnp.concatenate(
        [rows[0:TBLK // 2], rows[TBLK // 2:TBLK]], axis=1
    )


def _emb_body(xw_hbm, tok_hbm, pos_hbm, out_hbm, idx_v, stage_v, pos_v,
              gsem, osem):
    sid = lax.axis_index("s")
    wid = sid * NC + lax.axis_index("c")
    pltpu.sync_copy(xw_hbm.at[pl.ds(wid * NCHUNK, NCHUNK)], idx_v)

    @pl.when(sid == 0)
    def _():
        pltpu.sync_copy(pos_hbm, pos_v)

    plsc.subcore_barrier()
    row0 = wid * ROWS_PER_W

    def prefill(c):
        phase = lax.rem(c * CHUNK, T)
        pltpu.sync_copy(pos_v.at[pl.ds(phase, CHUNK)],
                        stage_v.at[lax.rem(c, NSLOT)])

    def fire_gadd(c):
        slot = lax.rem(c, NSLOT)
        pltpu.async_copy(tok_hbm.at[idx_v.at[c]], stage_v.at[slot],
                         gsem.at[slot], add=True)

    def write_desc(c):
        slot = lax.rem(c, NSLOT)
        return pltpu.make_async_copy(
            stage_v.at[slot],
            out_hbm.at[pl.ds(row0 + c * CHUNK, CHUNK)],
            osem.at[slot],
        )

    for c in range(5):
        prefill(c)
    for c in range(4):
        fire_gadd(c)

    def chunk_body(c, carry):
        slot = lax.rem(c, NSLOT)
        pltpu.make_async_copy(tok_hbm.at[idx_v.at[c]], stage_v.at[slot],
                              gsem.at[slot]).wait()
        write_desc(c).start()

        @pl.when(c >= 1)
        def _():
            write_desc(c - 1).wait()

        @pl.when(c + 5 < NCHUNK)
        def _():
            prefill(c + 5)

        @pl.when(c + 4 < NCHUNK)
        def _():
            fire_gadd(c + 4)

        return carry

    lax.fori_loop(0, NCHUNK, chunk_body, 0, unroll=False)
    write_desc(NCHUNK - 1).wait()


@jax.jit
def kernel(x, token_table, pos_table):
    nblk = pl.cdiv(V, TBLK)
    h = TBLK // 2
    xi = x.astype(jnp.int32).reshape(-1)
    k = xi & (TBLK - 1)
    pi = (xi & ~(TBLK - 1)) + 2 * (k & (h - 1)) + (k >= h)
    xw = pi.reshape(ROWS // CHUNK, CHUNK)
    tok2 = pl.pallas_call(
        _relayout_body,
        grid=(nblk,),
        in_specs=[pl.BlockSpec((D, TBLK), lambda j: (0, j))],
        out_specs=pl.BlockSpec((TBLK // 2, 128), lambda j: (j, 0)),
        out_shape=jax.ShapeDtypeStruct((nblk * (TBLK // 2), 128),
                                       jnp.float32),
    )(token_table.T)
    tok3 = tok2.reshape(nblk * TBLK, D)
    rr = jnp.arange(POS_ROWS) % T
    pos2 = pos_table[rr]
    mesh = plsc.VectorSubcoreMesh(core_axis_name="c", subcore_axis_name="s")
    run = pl.kernel(
        _emb_body,
        mesh=mesh,
        compiler_params=pltpu.CompilerParams(use_tc_tiling_on_sc=False),
        out_type=jax.ShapeDtypeStruct((ROWS, D), jnp.float32),
        scratch_types=[
            pltpu.VMEM((NCHUNK, CHUNK), jnp.int32),
            pltpu.VMEM((NSLOT, CHUNK, D), jnp.float32),
            pltpu.VMEM_SHARED((POS_ROWS, D), jnp.float32),
            pltpu.SemaphoreType.DMA((NSLOT,)),
            pltpu.SemaphoreType.DMA((NSLOT,)),
        ],
    )
    out = run(xw, tok3, pos2)
    return out.reshape(B, T, D)


# speedup vs baseline: 2.0505x; 1.0125x over previous
import jax
import jax.numpy as jnp
from jax import lax
from jax.experimental import pallas as pl
from jax.experimental.pallas import tpu as pltpu
from jax.experimental.pallas import tpu_sc as plsc

D = 64
T = 200
B = 4096
V = 1000000
NC, NS = 2, 16
NW = NC * NS
LANES = 16

ROWS = B * T
ROWS_PER_W = ROWS // NW
CHUNK = 128
NCHUNK = ROWS_PER_W // CHUNK
NSLOT = 6
POS_ROWS = T + CHUNK
TBLK = 32768


def _relayout_body(src_ref, dst_ref):
    eye = (lax.broadcasted_iota(jnp.int32, (D, D), 0)
           == lax.broadcasted_iota(jnp.int32, (D, D), 1)).astype(jnp.float32)
    rows = lax.dot_general(
        src_ref[...], eye,
        dimension_numbers=(((0,), (0,)), ((), ())),
        preferred_element_type=jnp.float32,
    )
    dst_ref[...] = jnp.concatenate(
        [rows[0:TBLK // 2], rows[TBLK // 2:TBLK]], axis=1
    )


def _emb_body(xw_hbm, tok_hbm, pos_hbm, out_hbm, idx_v, stage_v, pos_v,
              gsem, osem):
    sid = lax.axis_index("s")
    wid = sid * NC + lax.axis_index("c")
    pltpu.sync_copy(xw_hbm.at[pl.ds(wid * NCHUNK, NCHUNK)], idx_v)

    @pl.when(sid == 0)
    def _():
        pltpu.sync_copy(pos_hbm, pos_v)

    plsc.subcore_barrier()
    row0 = wid * ROWS_PER_W

    def prefill(c):
        phase = lax.rem(c * CHUNK, T)
        pltpu.sync_copy(pos_v.at[pl.ds(phase, CHUNK)],
                        stage_v.at[lax.rem(c, NSLOT)])

    def fire_gadd(c):
        slot = lax.rem(c, NSLOT)
        pltpu.async_copy(tok_hbm.at[idx_v.at[c]], stage_v.at[slot],
                         gsem.at[slot], add=True)

    def write_desc(c):
        slot = lax.rem(c, NSLOT)
        return pltpu.make_async_copy(
            stage_v.at[slot],
            out_hbm.at[pl.ds(row0 + c * CHUNK, CHUNK)],
            osem.at[slot],
        )

    for c in range(5):
        prefill(c)
    for c in range(4):
        fire_gadd(c)

    def chunk_body(c, carry):
        slot = lax.rem(c, NSLOT)
        pltpu.make_async_copy(tok_hbm.at[idx_v.at[c]], stage_v.at[slot],
                              gsem.at[slot]).wait()
        write_desc(c).start()

        @pl.when(c >= 1)
        def _():
            write_desc(c - 1).wait()

        @pl.when(c + 5 < NCHUNK)
        def _():
            prefill(c + 5)

        @pl.when(c + 4 < NCHUNK)
        def _():
            fire_gadd(c + 4)

        return carry

    lax.fori_loop(0, NCHUNK, chunk_body, 0, unroll=False)
    write_desc(NCHUNK - 1).wait()


@jax.jit
def kernel(x, token_table, pos_table):
    nblk = pl.cdiv(V, TBLK)
    h = TBLK // 2
    xi = x.astype(jnp.int32).reshape(-1)
    k = xi & (TBLK - 1)
    pi = (xi & ~(TBLK - 1)) + 2 * (k & (h - 1)) + (k >= h)
    xw = pi.reshape(ROWS // CHUNK, CHUNK)
    tok2 = pl.pallas_call(
        _relayout_body,
        grid=(nblk,),
        in_specs=[pl.BlockSpec((D, TBLK), lambda j: (0, j))],
        out_specs=pl.BlockSpec((TBLK // 2, 128), lambda j: (j, 0)),
        out_shape=jax.ShapeDtypeStruct((nblk * (TBLK // 2), 128),
                                       jnp.float32),
    )(token_table.T)
    tok3 = tok2.reshape(nblk * TBLK, D)
    rr = jnp.arange(POS_ROWS) % T
    pos2 = pos_table[rr]
    mesh = plsc.VectorSubcoreMesh(core_axis_name="c", subcore_axis_name="s")
    run = pl.kernel(
        _emb_body,
        mesh=mesh,
        compiler_params=pltpu.CompilerParams(use_tc_tiling_on_sc=False),
        out_type=jax.ShapeDtypeStruct((ROWS, D), jnp.float32),
        scratch_types=[
            pltpu.VMEM((NCHUNK, CHUNK), jnp.int32),
            pltpu.VMEM((NSLOT, CHUNK, D), jnp.float32),
            pltpu.VMEM_SHARED((POS_ROWS, D), jnp.float32),
            pltpu.SemaphoreType.DMA((NSLOT,)),
            pltpu.SemaphoreType.DMA((NSLOT,)),
        ],
    )
    out = run(xw, tok3, pos2)
    return out.reshape(B, T, D)
